# Initial kernel scaffold; baseline (speedup 1.0000x reference)
import jax
import jax.numpy as jnp
from jax.experimental import pallas as pl


def kernel(x, base_hvs):
    raise NotImplementedError("write your pallas kernel here")


# speedup vs baseline: 23.2339x; 23.2339x over previous
import jax
import jax.numpy as jnp
from jax.experimental import pallas as pl

_LEVELS = 100


def _tc_body(x_ref, tab_ref, o_ref):
    x = x_ref[...]
    Bt, D = x.shape
    L = _LEVELS
    H = tab_ref.shape[-1]
    xn = jnp.clip(x * (L - 1), 0.0, float(L - 1))
    low_f = jnp.floor(xn)
    a = xn - low_f
    low_i = low_f.astype(jnp.int32)
    high_i = jnp.minimum(low_i + 1, L - 1)
    lvl = jax.lax.broadcasted_iota(jnp.int32, (Bt, L), 1)
    acc = jnp.zeros((Bt, H), jnp.float32)
    for d in range(D):
        w = jnp.where(lvl == low_i[:, d:d + 1], 1.0 - a[:, d:d + 1], 0.0)
        w = w + jnp.where(lvl == high_i[:, d:d + 1], a[:, d:d + 1], 0.0)
        acc = acc + jnp.dot(w, tab_ref[d], preferred_element_type=jnp.float32)
    ss = jnp.sum(acc * acc, axis=1, keepdims=True)
    o_ref[...] = acc / jnp.sqrt(ss)


def kernel(x, base_hvs):
    if x.ndim == 1:
        x = x[None, :]
    B, D = x.shape
    _, L, H = base_hvs.shape
    Bt = 256
    return pl.pallas_call(
        _tc_body,
        grid=(B // Bt,),
        in_specs=[
            pl.BlockSpec((Bt, D), lambda i: (i, 0)),
            pl.BlockSpec((D, L, H), lambda i: (0, 0, 0)),
        ],
        out_specs=pl.BlockSpec((Bt, H), lambda i: (i, 0)),
        out_shape=jax.ShapeDtypeStruct((B, H), jnp.float32),
    )(x, base_hvs)
